# baseline (speedup 1.0000x reference)
import jax
import jax.numpy as jnp
from jax import lax
from jax.experimental import pallas as pl
from jax.experimental.pallas import tpu as pltpu
from jax.experimental.pallas import tpu_sc as plsc

VOCAB_ROWS = 1_000_000
BATCH = 16384
EMBED = 64
WIDE = 2 * EMBED
NUM_CORES = 2
NUM_SUBCORES = 16
NUM_TILES = NUM_CORES * NUM_SUBCORES
B_PER_TILE = BATCH // NUM_TILES
PAD_BLOCK_ROWS = 8000


def _widen_kernel(w_ref, o_ref):
    x = w_ref[...]
    o_ref[:, :EMBED] = x
    o_ref[:, EMBED:] = x


def _widen(W):
    return pl.pallas_call(
        _widen_kernel,
        grid=(VOCAB_ROWS // PAD_BLOCK_ROWS,),
        in_specs=[pl.BlockSpec((PAD_BLOCK_ROWS, EMBED), lambda i: (i, 0))],
        out_specs=pl.BlockSpec((PAD_BLOCK_ROWS, WIDE), lambda i: (i, 0)),
        out_shape=jax.ShapeDtypeStruct((VOCAB_ROWS, WIDE), W.dtype),
        compiler_params=pltpu.CompilerParams(
            dimension_semantics=("parallel",),
        ),
    )(W)


def kernel(inputs, W):
    idx = inputs.reshape((BATCH,))
    Wp = _widen(W)

    mesh = plsc.VectorSubcoreMesh(core_axis_name="c", subcore_axis_name="s")

    @pl.kernel(
        out_type=jax.ShapeDtypeStruct((BATCH, WIDE), W.dtype),
        mesh=mesh,
        scratch_types=[
            pltpu.VMEM((B_PER_TILE,), jnp.int32),
            pltpu.VMEM((B_PER_TILE, WIDE), jnp.float32),
            pltpu.SemaphoreType.DMA,
        ],
    )
    def gather_kernel(table_hbm, idx_hbm, out_hbm, idx_v, rows_v, sem):
        wid = lax.axis_index("s") * NUM_CORES + lax.axis_index("c")
        base = wid * B_PER_TILE
        pltpu.sync_copy(idx_hbm.at[pl.ds(base, B_PER_TILE)], idx_v)
        pltpu.async_copy(table_hbm.at[idx_v], rows_v, sem).wait()
        pltpu.sync_copy(rows_v, out_hbm.at[pl.ds(base, B_PER_TILE)])

    big = gather_kernel(Wp, idx)
    return big[:, :EMBED]


# speedup vs baseline: 1.0085x; 1.0085x over previous
import jax
import jax.numpy as jnp
from jax import lax
from jax.experimental import pallas as pl
from jax.experimental.pallas import tpu as pltpu
from jax.experimental.pallas import tpu_sc as plsc

VOCAB_ROWS = 1_000_000
BATCH = 16384
EMBED = 64
WIDE = 2 * EMBED
NUM_CORES = 2
NUM_SUBCORES = 16
NUM_TILES = NUM_CORES * NUM_SUBCORES
B_PER_TILE = BATCH // NUM_TILES
PAD_BLOCK_ROWS = 8000


def _widen_kernel(w_ref, o_ref):
    o_ref[:, :EMBED] = w_ref[...]


def _widen(W):
    return pl.pallas_call(
        _widen_kernel,
        grid=(VOCAB_ROWS // PAD_BLOCK_ROWS,),
        in_specs=[pl.BlockSpec((PAD_BLOCK_ROWS, EMBED), lambda i: (i, 0))],
        out_specs=pl.BlockSpec((PAD_BLOCK_ROWS, WIDE), lambda i: (i, 0)),
        out_shape=jax.ShapeDtypeStruct((VOCAB_ROWS, WIDE), W.dtype),
        compiler_params=pltpu.CompilerParams(
            dimension_semantics=("parallel",),
        ),
    )(W)


def kernel(inputs, W):
    idx = inputs.reshape((BATCH,))
    Wp = _widen(W)

    mesh = plsc.VectorSubcoreMesh(core_axis_name="c", subcore_axis_name="s")

    @pl.kernel(
        out_type=jax.ShapeDtypeStruct((BATCH, WIDE), W.dtype),
        mesh=mesh,
        scratch_types=[
            pltpu.VMEM((B_PER_TILE,), jnp.int32),
            pltpu.VMEM((B_PER_TILE, WIDE), jnp.float32),
            pltpu.SemaphoreType.DMA,
        ],
    )
    def gather_kernel(table_hbm, idx_hbm, out_hbm, idx_v, rows_v, sem):
        wid = lax.axis_index("s") * NUM_CORES + lax.axis_index("c")
        base = wid * B_PER_TILE
        pltpu.sync_copy(idx_hbm.at[pl.ds(base, B_PER_TILE)], idx_v)
        pltpu.async_copy(table_hbm.at[idx_v], rows_v, sem).wait()
        pltpu.sync_copy(rows_v, out_hbm.at[pl.ds(base, B_PER_TILE)])

    big = gather_kernel(Wp, idx)
    return big[:, :EMBED]


# speedup vs baseline: 1.8852x; 1.8694x over previous
import jax
import jax.numpy as jnp
from jax import lax
from jax.experimental import pallas as pl
from jax.experimental.pallas import tpu as pltpu
from jax.experimental.pallas import tpu_sc as plsc

BATCH = 16384
EMBED = 64
NUM_CORES = 2
NUM_SUBCORES = 16
NUM_TILES = NUM_CORES * NUM_SUBCORES
B_PER_TILE = BATCH // NUM_TILES
LANES = 16
N_CHUNKS = B_PER_TILE // LANES


def kernel(inputs, W):
    idx = inputs.reshape((BATCH,))

    mesh = plsc.VectorSubcoreMesh(core_axis_name="c", subcore_axis_name="s")

    @pl.kernel(
        out_type=jax.ShapeDtypeStruct((BATCH, EMBED), W.dtype),
        mesh=mesh,
        scratch_types=[
            pltpu.VMEM((B_PER_TILE,), jnp.int32),
            pltpu.VMEM((B_PER_TILE, EMBED), jnp.float32),
            pltpu.SemaphoreType.DMA,
            pltpu.SemaphoreType.DMA,
        ],
        compiler_params=pltpu.CompilerParams(needs_layout_passes=False),
    )
    def gather_kernel(table_hbm, idx_hbm, out_hbm, idx_v, rows_v, sem_i, sem):
        wid = lax.axis_index("s") * NUM_CORES + lax.axis_index("c")
        base = wid * B_PER_TILE
        pltpu.async_copy(idx_hbm.at[pl.ds(base, B_PER_TILE)], idx_v, sem_i).wait()

        lane = lax.broadcasted_iota(jnp.int32, (LANES,), 0)

        @pl.loop(0, N_CHUNKS)
        def _(c):
            chunk = idx_v[pl.ds(c * LANES, LANES)]
            for j in range(LANES):
                i = jnp.sum(jnp.where(lane == j, chunk, 0))
                pltpu.make_async_copy(
                    table_hbm.at[pl.ds(i, 1)],
                    rows_v.at[pl.ds(c * LANES + j, 1)],
                    sem,
                ).start()

        @pl.loop(0, B_PER_TILE)
        def _(b):
            pltpu.make_async_copy(
                table_hbm.at[pl.ds(0, 1)],
                rows_v.at[pl.ds(b, 1)],
                sem,
            ).wait()

        pltpu.sync_copy(rows_v, out_hbm.at[pl.ds(base, B_PER_TILE)])

    return gather_kernel(W, idx)
